# Initial kernel scaffold; baseline (speedup 1.0000x reference)
import functools

import jax
import jax.numpy as jnp
from jax import lax
from jax.experimental import pallas as pl
from jax.experimental.pallas import tpu as pltpu
from jax.experimental.pallas import tpu_sc as plsc

NC = 2
NS = 16
LN = 16


def _pick_batch(ew: int) -> int:
    for b in (128, 112, 96, 80, 64, 48, 32, 16):
        if ew % b == 0:
            return b
    raise ValueError(f"per-subcore edge count {ew} not a multiple of 16")


def _mesh():
    return plsc.VectorSubcoreMesh(core_axis_name="c", subcore_axis_name="s",
                                  num_cores=NC, num_subcores=NS)


def _sc_count(dst, etype, n, r):
    e = dst.shape[0]
    nw = NC * NS
    ew = e // nw
    bsz = _pick_batch(ew)
    nb = ew // bsz
    rn = n * r
    sl_sz = rn // NS

    @functools.partial(
        pl.kernel,
        out_type=jax.ShapeDtypeStruct((NC, rn), jnp.float32),
        mesh=_mesh(),
        scratch_types=[
            pltpu.VMEM((bsz,), jnp.int32),
            pltpu.VMEM((bsz,), jnp.int32),
            pltpu.VMEM((bsz,), jnp.int32),
            pltpu.VMEM((bsz,), jnp.float32),
            pltpu.VMEM((sl_sz,), jnp.float32),
            pltpu.VMEM_SHARED((rn,), jnp.float32),
        ],
    )
    def cnt_kernel(dst_hbm, type_hbm, out_hbm, dst_v, type_v, nidx_v, ones_v,
                   z_v, cnt_sh):
        c = lax.axis_index("c")
        s = lax.axis_index("s")
        wid = s * NC + c

        def fill(i, carry):
            ones_v[pl.ds(i * LN, LN)] = jnp.full((LN,), 1.0, jnp.float32)
            return carry
        lax.fori_loop(0, bsz // LN, fill, None)

        def zfill(i, carry):
            z_v[pl.ds(i * LN, LN)] = jnp.zeros((LN,), jnp.float32)
            return carry
        lax.fori_loop(0, sl_sz // LN, zfill, None)
        pltpu.sync_copy(z_v, cnt_sh.at[pl.ds(s * sl_sz, sl_sz)])
        plsc.subcore_barrier()

        base0 = wid * ew

        def batch(bi, carry):
            base = base0 + bi * bsz
            pltpu.sync_copy(dst_hbm.at[pl.ds(base, bsz)], dst_v)
            pltpu.sync_copy(type_hbm.at[pl.ds(base, bsz)], type_v)
            for k in range(bsz // LN):
                sl = pl.ds(k * LN, LN)
                nidx_v[sl] = type_v[sl] * n + dst_v[sl]
            pltpu.sync_copy(ones_v, cnt_sh.at[nidx_v], add=True)
            return carry
        lax.fori_loop(0, nb, batch, None)

        plsc.subcore_barrier()
        pltpu.sync_copy(cnt_sh.at[pl.ds(s * sl_sz, sl_sz)],
                        out_hbm.at[c, pl.ds(s * sl_sz, sl_sz)])

    return cnt_kernel(dst, etype)


def _sc_agg(h_tab, norm, src, dst, etype, n, d):
    e = src.shape[0]
    nw = NC * NS
    ew = e // nw
    bsz = _pick_batch(ew)
    nb = ew // bsz
    rows_per_tile = n // NS
    zrows = rows_per_tile
    while zrows * d * 4 > 64 * 1024:
        zrows = zrows // 5 if zrows % 5 == 0 else zrows // 2
    nz = rows_per_tile // zrows

    @functools.partial(
        pl.kernel,
        out_type=jax.ShapeDtypeStruct((NC, n, d), jnp.float32),
        mesh=_mesh(),
        scratch_types=[
            pltpu.VMEM((bsz,), jnp.int32),
            pltpu.VMEM((bsz,), jnp.int32),
            pltpu.VMEM((bsz,), jnp.int32),
            pltpu.VMEM((bsz,), jnp.int32),
            pltpu.VMEM((bsz,), jnp.int32),
            pltpu.VMEM((bsz,), jnp.float32),
            pltpu.VMEM((bsz, d), jnp.float32),
            pltpu.VMEM((0,), jnp.float32),
            pltpu.VMEM((1, d), jnp.float32),
            pltpu.VMEM_SHARED((n, d), jnp.float32),
        ],
    )
    def agg_kernel(h_hbm, nrm_hbm, src_hbm, dst_hbm, type_hbm, out_hbm,
                   src_v, dst_v, type_v, eidx_v, nidx_v, w_v, rows_v, ph_v,
                   ph2_v, acc_sh):
        c = lax.axis_index("c")
        s = lax.axis_index("s")
        wid = s * NC + c

        def zfill(i, carry):
            row = i // (d // LN)
            col = i % (d // LN)
            rows_v[row, pl.ds(col * LN, LN)] = jnp.zeros((LN,), jnp.float32)
            return carry
        lax.fori_loop(0, bsz * (d // LN), zfill, None)

        def zcopy(i, carry):
            pltpu.sync_copy(
                rows_v, acc_sh.at[pl.ds(s * rows_per_tile + i * bsz, bsz)])
            return carry
        lax.fori_loop(0, rows_per_tile // bsz, zcopy, None)
        ztail = rows_per_tile % bsz
        if ztail:
            pltpu.sync_copy(
                rows_v.at[pl.ds(0, ztail)],
                acc_sh.at[pl.ds(s * rows_per_tile
                                + (rows_per_tile // bsz) * bsz, ztail)])
        plsc.subcore_barrier()

        base0 = wid * ew

        def batch(bi, carry):
            base = base0 + bi * bsz
            pltpu.sync_copy(src_hbm.at[pl.ds(base, bsz)], src_v)
            pltpu.sync_copy(dst_hbm.at[pl.ds(base, bsz)], dst_v)
            pltpu.sync_copy(type_hbm.at[pl.ds(base, bsz)], type_v)
            for k in range(bsz // LN):
                sl = pl.ds(k * LN, LN)
                t = type_v[sl]
                eidx_v[sl] = t * n + src_v[sl]
                nidx_v[sl] = t * n + dst_v[sl]
            pltpu.sync_copy(nrm_hbm.at[nidx_v], w_v)
            pltpu.sync_copy(h_hbm.at[eidx_v], rows_v)

            def scale(j, carry2):
                wb = plsc.load_gather(w_v, [jnp.full((LN,), j, jnp.int32)])
                for cc in range(d // LN):
                    sl2 = pl.ds(cc * LN, LN)
                    rows_v[j, sl2] = rows_v[j, sl2] * wb
                return carry2
            lax.fori_loop(0, bsz, scale, None)

            pltpu.sync_copy(rows_v, acc_sh.at[dst_v], add=True)
            return carry
        lax.fori_loop(0, nb, batch, None)

        plsc.subcore_barrier()
        pltpu.sync_copy(acc_sh.at[pl.ds(s * rows_per_tile, rows_per_tile)],
                        out_hbm.at[c, pl.ds(s * rows_per_tile, rows_per_tile)])

    return agg_kernel(h_tab, norm, src, dst, etype)


def _rows_block(n):
    for b in (1000, 2000, 500, 200, 1024, 512, 256, 128):
        if n % b == 0:
            return b
    return n


def _relmat_body(x_ref, w_ref, o_ref):
    o_ref[...] = jnp.dot(x_ref[...], w_ref[0],
                         preferred_element_type=jnp.float32)


def _tc_relmat(xin, w_rel):
    n, d_in = xin.shape
    r, _, d_out = w_rel.shape
    blk = _rows_block(n)
    nbk = n // blk
    return pl.pallas_call(
        _relmat_body,
        grid=(r, nbk),
        in_specs=[
            pl.BlockSpec((blk, d_in), lambda ri, i: (i, 0)),
            pl.BlockSpec((1, d_in, d_out), lambda ri, i: (ri, 0, 0)),
        ],
        out_specs=pl.BlockSpec((blk, d_out), lambda ri, i: (ri * nbk + i, 0)),
        out_shape=jax.ShapeDtypeStruct((r * n, d_out), jnp.float32),
    )(xin, w_rel)


def _norm_body(c_ref, o_ref):
    o_ref[...] = 1.0 / jnp.maximum(c_ref[0] + c_ref[1], 1.0)


def _tc_norm(cnt_part):
    nc, rn = cnt_part.shape
    c3 = cnt_part.reshape(nc, rn // 128, 128)
    out = pl.pallas_call(
        _norm_body,
        out_shape=jax.ShapeDtypeStruct((rn // 128, 128), jnp.float32),
    )(c3)
    return out.reshape(rn)


def _combine_body_relu(agg_ref, x_ref, w_ref, b_ref, o_ref):
    v = (agg_ref[0] + agg_ref[1] + b_ref[...]
         + jnp.dot(x_ref[...], w_ref[...], preferred_element_type=jnp.float32))
    o_ref[...] = jnp.maximum(v, 0.0)


def _combine_body(agg_ref, x_ref, w_ref, b_ref, o_ref):
    o_ref[...] = (agg_ref[0] + agg_ref[1] + b_ref[...]
                  + jnp.dot(x_ref[...], w_ref[...],
                            preferred_element_type=jnp.float32))


def _tc_combine(agg, xin, w_root, b, relu):
    n, d_in = xin.shape
    d_out = w_root.shape[1]
    blk = _rows_block(n)
    nbk = n // blk
    body = _combine_body_relu if relu else _combine_body
    return pl.pallas_call(
        body,
        grid=(nbk,),
        in_specs=[
            pl.BlockSpec((NC, blk, d_out), lambda i: (0, i, 0)),
            pl.BlockSpec((blk, d_in), lambda i: (i, 0)),
            pl.BlockSpec((d_in, d_out), lambda i: (0, 0)),
            pl.BlockSpec((1, d_out), lambda i: (0, 0)),
        ],
        out_specs=pl.BlockSpec((blk, d_out), lambda i: (i, 0)),
        out_shape=jax.ShapeDtypeStruct((n, d_out), jnp.float32),
    )(agg, xin, w_root, b.reshape(1, d_out))


def kernel(x, edge_index, edge_type, W_rel1, W_root1, b1, W_rel2, W_root2, b2):
    n, _ = x.shape
    r = W_rel1.shape[0]
    d_hid = W_rel1.shape[2]
    d_out = W_rel2.shape[2]
    src = edge_index[0]
    dst = edge_index[1]

    cnt_part = _sc_count(dst, edge_type, n, r)
    norm = _tc_norm(cnt_part)

    h_tab1 = _tc_relmat(x, W_rel1)
    agg1 = _sc_agg(h_tab1, norm, src, dst, edge_type, n, d_hid)
    h1 = _tc_combine(agg1, x, W_root1, b1, relu=True)

    h_tab2 = _tc_relmat(h1, W_rel2)
    agg2 = _sc_agg(h_tab2, norm, src, dst, edge_type, n, d_out)
    out = _tc_combine(agg2, h1, W_root2, b2, relu=False)
    return out


# speedup vs baseline: 11.4410x; 11.4410x over previous
import functools

import jax
import jax.numpy as jnp
from jax import lax
from jax.experimental import pallas as pl
from jax.experimental.pallas import tpu as pltpu
from jax.experimental.pallas import tpu_sc as plsc

NC = 2
NS = 16
LN = 16


def _pick_batch(ew: int) -> int:
    for b in (128, 112, 96, 80, 64, 48, 32, 16):
        if ew % b == 0:
            return b
    raise ValueError(f"per-subcore edge count {ew} not a multiple of 16")


def _mesh():
    return plsc.VectorSubcoreMesh(core_axis_name="c", subcore_axis_name="s",
                                  num_cores=NC, num_subcores=NS)


def _sc_count(dst, etype, n, r):
    e = dst.shape[0]
    nw = NC * NS
    ew = e // nw
    bsz = _pick_batch(ew)
    nb = ew // bsz
    rn = n * r
    sl_sz = rn // NS

    @functools.partial(
        pl.kernel,
        out_type=jax.ShapeDtypeStruct((NC * rn,), jnp.float32),
        mesh=_mesh(),
        scratch_types=[
            pltpu.VMEM((bsz,), jnp.int32),
            pltpu.VMEM((bsz,), jnp.int32),
            pltpu.VMEM((bsz,), jnp.int32),
            pltpu.VMEM((bsz,), jnp.float32),
            pltpu.VMEM((sl_sz,), jnp.float32),
            pltpu.VMEM_SHARED((rn,), jnp.float32),
        ],
    )
    def cnt_kernel(dst_hbm, type_hbm, out_hbm, dst_v, type_v, nidx_v, ones_v,
                   z_v, cnt_sh):
        c = lax.axis_index("c")
        s = lax.axis_index("s")
        wid = s * NC + c

        def fill(i, carry):
            ones_v[pl.ds(i * LN, LN)] = jnp.full((LN,), 1.0, jnp.float32)
            return carry
        lax.fori_loop(0, bsz // LN, fill, None)

        def zfill(i, carry):
            z_v[pl.ds(i * LN, LN)] = jnp.zeros((LN,), jnp.float32)
            return carry
        lax.fori_loop(0, sl_sz // LN, zfill, None)
        pltpu.sync_copy(z_v, cnt_sh.at[pl.ds(s * sl_sz, sl_sz)])
        plsc.subcore_barrier()

        base0 = wid * ew

        def batch(bi, carry):
            base = base0 + bi * bsz
            pltpu.sync_copy(dst_hbm.at[pl.ds(base, bsz)], dst_v)
            pltpu.sync_copy(type_hbm.at[pl.ds(base, bsz)], type_v)
            for k in range(bsz // LN):
                sl = pl.ds(k * LN, LN)
                nidx_v[sl] = type_v[sl] * n + dst_v[sl]
            pltpu.sync_copy(ones_v, cnt_sh.at[nidx_v], add=True)
            return carry
        lax.fori_loop(0, nb, batch, None)

        plsc.subcore_barrier()
        pltpu.sync_copy(cnt_sh.at[pl.ds(s * sl_sz, sl_sz)], z_v)
        pltpu.sync_copy(z_v, out_hbm.at[pl.ds(c * rn + s * sl_sz, sl_sz)])

    return cnt_kernel(dst, etype)


def _sc_agg(h_tab, norm, src, dst, etype, n, d):
    e = src.shape[0]
    nw = NC * NS
    ew = e // nw
    bsz = _pick_batch(ew)
    nb = ew // bsz
    rows_per_tile = n // NS

    @functools.partial(
        pl.kernel,
        out_type=jax.ShapeDtypeStruct((NC, n, d), jnp.float32),
        mesh=_mesh(),
        scratch_types=[
            pltpu.VMEM((bsz,), jnp.int32),
            pltpu.VMEM((bsz,), jnp.int32),
            pltpu.VMEM((bsz,), jnp.int32),
            pltpu.VMEM((bsz,), jnp.int32),
            pltpu.VMEM((bsz,), jnp.int32),
            pltpu.VMEM((bsz,), jnp.float32),
            pltpu.VMEM((bsz, d), jnp.float32),
            pltpu.VMEM_SHARED((n, d), jnp.float32),
        ],
    )
    def agg_kernel(h_hbm, nrm_hbm, src_hbm, dst_hbm, type_hbm, out_hbm,
                   src_v, dst_v, type_v, eidx_v, nidx_v, w_v, rows_v, acc_sh):
        c = lax.axis_index("c")
        s = lax.axis_index("s")
        wid = s * NC + c

        def zfill(i, carry):
            row = i // (d // LN)
            col = i % (d // LN)
            rows_v[row, pl.ds(col * LN, LN)] = jnp.zeros((LN,), jnp.float32)
            return carry
        lax.fori_loop(0, bsz * (d // LN), zfill, None)

        def zcopy(i, carry):
            pltpu.sync_copy(
                rows_v, acc_sh.at[pl.ds(s * rows_per_tile + i * bsz, bsz)])
            return carry
        lax.fori_loop(0, rows_per_tile // bsz, zcopy, None)
        ztail = rows_per_tile % bsz
        if ztail:
            pltpu.sync_copy(
                rows_v.at[pl.ds(0, ztail)],
                acc_sh.at[pl.ds(s * rows_per_tile
                                + (rows_per_tile // bsz) * bsz, ztail)])
        plsc.subcore_barrier()

        base0 = wid * ew

        def batch(bi, carry):
            base = base0 + bi * bsz
            pltpu.sync_copy(src_hbm.at[pl.ds(base, bsz)], src_v)
            pltpu.sync_copy(dst_hbm.at[pl.ds(base, bsz)], dst_v)
            pltpu.sync_copy(type_hbm.at[pl.ds(base, bsz)], type_v)
            for k in range(bsz // LN):
                sl = pl.ds(k * LN, LN)
                t = type_v[sl]
                eidx_v[sl] = t * n + src_v[sl]
                nidx_v[sl] = t * n + dst_v[sl]
            pltpu.sync_copy(nrm_hbm.at[nidx_v], w_v)
            pltpu.sync_copy(h_hbm.at[eidx_v], rows_v)

            def scale(k, carry2):
                wv = w_v[pl.ds(k * LN, LN)]
                for jj in range(LN):
                    j = k * LN + jj
                    wb = jnp.full((LN,), wv[jj], jnp.float32)
                    for cc in range(d // LN):
                        sl2 = pl.ds(cc * LN, LN)
                        rows_v[j, sl2] = rows_v[j, sl2] * wb
                return carry2
            lax.fori_loop(0, bsz // LN, scale, None)

            pltpu.sync_copy(rows_v, acc_sh.at[dst_v], add=True)
            return carry
        lax.fori_loop(0, nb, batch, None)

        plsc.subcore_barrier()
        g_per = (n // 8) // NS
        rem = (n // 8) - g_per * NS
        row0 = s * (g_per * 8)
        pltpu.sync_copy(acc_sh.at[pl.ds(row0, g_per * 8)],
                        out_hbm.at[c, pl.ds(row0, g_per * 8)])
        if rem:
            @pl.when(s == NS - 1)
            def _tail_copy():
                r0 = NS * g_per * 8
                pltpu.sync_copy(acc_sh.at[pl.ds(r0, rem * 8)],
                                out_hbm.at[c, pl.ds(r0, rem * 8)])

    return agg_kernel(h_tab, norm, src, dst, etype)


def _rows_block(n):
    for b in (1000, 2000, 500, 200, 1024, 512, 256, 128):
        if n % b == 0:
            return b
    return n


def _relmat_body(x_ref, w_ref, o_ref):
    o_ref[...] = jnp.dot(x_ref[...], w_ref[0],
                         preferred_element_type=jnp.float32)


def _tc_relmat(xin, w_rel):
    n, d_in = xin.shape
    r, _, d_out = w_rel.shape
    blk = _rows_block(n)
    nbk = n // blk
    return pl.pallas_call(
        _relmat_body,
        grid=(r, nbk),
        in_specs=[
            pl.BlockSpec((blk, d_in), lambda ri, i: (i, 0)),
            pl.BlockSpec((1, d_in, d_out), lambda ri, i: (ri, 0, 0)),
        ],
        out_specs=pl.BlockSpec((blk, d_out), lambda ri, i: (ri * nbk + i, 0)),
        out_shape=jax.ShapeDtypeStruct((r * n, d_out), jnp.float32),
    )(xin, w_rel)


def _norm_body(c_ref, o_ref):
    o_ref[...] = 1.0 / jnp.maximum(c_ref[0] + c_ref[1], 1.0)


def _tc_norm(cnt_part):
    rn = cnt_part.shape[0] // NC
    c3 = cnt_part.reshape(NC, rn // 128, 128)
    out = pl.pallas_call(
        _norm_body,
        out_shape=jax.ShapeDtypeStruct((rn // 128, 128), jnp.float32),
    )(c3)
    return out.reshape(rn)


def _combine_body_relu(agg_ref, x_ref, w_ref, b_ref, o_ref):
    v = (agg_ref[0] + agg_ref[1] + b_ref[...]
         + jnp.dot(x_ref[...], w_ref[...], preferred_element_type=jnp.float32))
    o_ref[...] = jnp.maximum(v, 0.0)


def _combine_body(agg_ref, x_ref, w_ref, b_ref, o_ref):
    o_ref[...] = (agg_ref[0] + agg_ref[1] + b_ref[...]
                  + jnp.dot(x_ref[...], w_ref[...],
                            preferred_element_type=jnp.float32))


def _tc_combine(agg, xin, w_root, b, relu):
    n, d_in = xin.shape
    d_out = w_root.shape[1]
    blk = _rows_block(n)
    nbk = n // blk
    body = _combine_body_relu if relu else _combine_body
    return pl.pallas_call(
        body,
        grid=(nbk,),
        in_specs=[
            pl.BlockSpec((NC, blk, d_out), lambda i: (0, i, 0)),
            pl.BlockSpec((blk, d_in), lambda i: (i, 0)),
            pl.BlockSpec((d_in, d_out), lambda i: (0, 0)),
            pl.BlockSpec((1, d_out), lambda i: (0, 0)),
        ],
        out_specs=pl.BlockSpec((blk, d_out), lambda i: (i, 0)),
        out_shape=jax.ShapeDtypeStruct((n, d_out), jnp.float32),
    )(agg, xin, w_root, b.reshape(1, d_out))


def kernel(x, edge_index, edge_type, W_rel1, W_root1, b1, W_rel2, W_root2, b2):
    n, _ = x.shape
    r = W_rel1.shape[0]
    d_hid = W_rel1.shape[2]
    d_out = W_rel2.shape[2]
    src = edge_index[0]
    dst = edge_index[1]

    cnt_part = _sc_count(dst, edge_type, n, r)
    norm = _tc_norm(cnt_part)

    h_tab1 = _tc_relmat(x, W_rel1)
    agg1 = _sc_agg(h_tab1, norm, src, dst, edge_type, n, d_hid)
    h1 = _tc_combine(agg1, x, W_root1, b1, relu=True)

    h_tab2 = _tc_relmat(h1, W_rel2)
    agg2 = _sc_agg(h_tab2, norm, src, dst, edge_type, n, d_out)
    out = _tc_combine(agg2, h1, W_root2, b2, relu=False)
    return out
